# Initial kernel scaffold; baseline (speedup 1.0000x reference)
import jax
import jax.numpy as jnp
from jax.experimental import pallas as pl


def kernel(x, W1s, W1d, a1, b1, W2s, W2d, a2, b2, Wr1, br1, Wr2, br2, edge_index):
    raise NotImplementedError("write your pallas kernel here")


# speedup vs baseline: 16.3123x; 16.3123x over previous
import functools

import jax
import jax.numpy as jnp
from jax import lax
from jax.experimental import pallas as pl
from jax.experimental.pallas import tpu as pltpu
from jax.experimental.pallas import tpu_sc as plsc

N = 10000
E = 160000
B = 64
NSUB = 16
NCORE = 2
NBLK = E // B
RCH = 40
NRCH = N // RCH

@functools.lru_cache(maxsize=None)
def _mesh():
    return plsc.VectorSubcoreMesh(
        core_axis_name="c", subcore_axis_name="s",
        num_cores=NCORE, num_subcores=NSUB)


@functools.lru_cache(maxsize=None)
def _make_edge_kernel(nheads, stacked):
    nch = 8
    alen = 256 if stacked else 64

    @functools.partial(
        pl.kernel,
        out_type=(
            jax.ShapeDtypeStruct((NCORE, N, 128), jnp.float32),
            jax.ShapeDtypeStruct((NCORE * NSUB * nheads * N,), jnp.float32),
        ),
        mesh=_mesh(),
        compiler_params=pltpu.CompilerParams(needs_layout_passes=False),
        scratch_types=(
            pltpu.VMEM((B,), jnp.int32),
            pltpu.VMEM((B,), jnp.int32),
            pltpu.VMEM((B,), jnp.int32),
            pltpu.VMEM((B + 16,), jnp.int32),
            pltpu.VMEM((B, 128), jnp.float32),
            pltpu.VMEM((B, 128), jnp.float32),
            pltpu.VMEM((B, 128), jnp.float32),
            pltpu.VMEM((nheads * N + 16,), jnp.float32),
            pltpu.VMEM((alen,), jnp.float32),
            pltpu.SemaphoreType.DMA,
            pltpu.SemaphoreType.DMA,
            pltpu.VMEM_SHARED((N, 128), jnp.float32),
        ),
    )
    def k(fs_hbm, fd_hbm, src_hbm, dst_hbm, a_hbm, msg_out, den_out,
          sidx, didx, didx2, didxp, fsrows, fdrows, msgbuf, denloc, avm,
          sem1, sem2, msg_sh):
        c = lax.axis_index("c")
        s = lax.axis_index("s")
        zero = jnp.zeros((16,), jnp.float32)
        lane = lax.iota(jnp.int32, 16)

        def zb(r, carry):
            for kk in range(nch):
                msgbuf[r, pl.ds(kk * 16, 16)] = zero
            return carry
        lax.fori_loop(0, B, zb, 0)

        def zd(r, carry):
            denloc[pl.ds(r * 16, 16)] = zero
            return carry
        lax.fori_loop(0, nheads * N // 16 + 1, zd, 0)
        cntz = (NRCH - s + NSUB - 1) // NSUB

        def zcopy(t, carry):
            o = pl.multiple_of((s + t * NSUB) * RCH, 8)
            pltpu.sync_copy(msgbuf.at[pl.ds(0, RCH)],
                            msg_sh.at[pl.ds(o, RCH)])
            return carry
        lax.fori_loop(0, cntz, zcopy, 0)

        pltpu.sync_copy(a_hbm, avm)
        avs = []
        for kk in range(4 * nheads):
            if stacked:
                lo = avm[pl.ds(kk * 16, 16)]
                hi = avm[pl.ds(128 + kk * 16, 16)]
                avs.append(jnp.where(c == 0, lo, hi))
            else:
                avs.append(avm[pl.ds(kk * 16, 16)])

        plsc.subcore_barrier()

        if stacked:
            cnt = (NBLK - s + NSUB - 1) // NSUB
        else:
            half = NBLK // NCORE
            cnt = (half - s + NSUB - 1) // NSUB

        def blk_body(i, carry):
            if stacked:
                blk = s + i * NSUB
            else:
                blk = c * (NBLK // NCORE) + s + i * NSUB
            base = pl.multiple_of(blk * B, B)
            pltpu.sync_copy(src_hbm.at[pl.ds(base, B)], sidx)
            pltpu.sync_copy(dst_hbm.at[pl.ds(base, B)], didx)
            if stacked:
                off = c * N
                for kk in range(B // 16):
                    sidx[pl.ds(kk * 16, 16)] = sidx[pl.ds(kk * 16, 16)] + off
                    didx2[pl.ds(kk * 16, 16)] = didx[pl.ds(kk * 16, 16)] + off
                pltpu.async_copy(fs_hbm.at[sidx], fsrows, sem1).wait()
                pltpu.async_copy(fd_hbm.at[didx2], fdrows, sem2).wait()
            else:
                pltpu.async_copy(fs_hbm.at[sidx], fsrows, sem1).wait()
                pltpu.async_copy(fd_hbm.at[didx], fdrows, sem2).wait()
            for kk in range(B // 16):
                didxp[pl.ds(kk * 16, 16)] = didx[pl.ds(kk * 16, 16)]

            def ebody(r, ecarry):
                fsl = [fsrows[r, pl.ds(kk * 16, 16)] for kk in range(nch)]
                fdl = [fdrows[r, pl.ds(kk * 16, 16)]
                       for kk in range(4 * nheads)]
                di = didxp[pl.ds(r, 16)][0]
                exvs = []
                for j in range(nheads):
                    acc = zero
                    for kk in range(4):
                        q = j * 4 + kk
                        ev = fsl[q] + fdl[q]
                        ev = jnp.maximum(ev, 0.2 * ev)
                        acc = acc + ev * avs[q]
                    sco = jnp.sum(acc)
                    exv = jnp.exp(jnp.full((16,), sco, jnp.float32))
                    exvs.append(exv)
                    p = j * N + di
                    dv = denloc[pl.ds(p, 16)]
                    denloc[pl.ds(p, 16)] = dv + jnp.where(lane == 0, exv,
                                                          zero)
                for q in range(nch):
                    msgbuf[r, pl.ds(q * 16, 16)] = (
                        fsl[q] * exvs[min(q // 4, nheads - 1)])
                return ecarry
            lax.fori_loop(0, B, ebody, 0)

            pltpu.sync_copy(msgbuf, msg_sh.at[didx], add=True)
            return carry
        lax.fori_loop(0, cnt, blk_body, 0)

        doff = pl.multiple_of((c * NSUB + s) * nheads * N, 8)
        pltpu.sync_copy(denloc.at[pl.ds(0, nheads * N)],
                        den_out.at[pl.ds(doff, nheads * N)])

        plsc.subcore_barrier()

        def dcopy(t, carry):
            o = pl.multiple_of((s + t * NSUB) * RCH, 8)
            pltpu.sync_copy(msg_sh.at[pl.ds(o, RCH)], msgbuf.at[pl.ds(0, RCH)])
            pltpu.sync_copy(msgbuf.at[pl.ds(0, RCH)],
                            msg_out.at[c, pl.ds(o, RCH)])
            return carry
        lax.fori_loop(0, cntz, dcopy, 0)

    return k


_RB = 400


def _proj1(x, W1s, W1d):
    nb = N // _RB

    def body(x_ref, ws_ref, wd_ref, fs_ref, fd_ref):
        xb = x_ref[...]
        fs_ref[...] = jnp.dot(xb, ws_ref[...], preferred_element_type=jnp.float32)
        fd_ref[...] = jnp.dot(xb, wd_ref[...], preferred_element_type=jnp.float32)

    return pl.pallas_call(
        body,
        grid=(nb, 2),
        in_specs=[
            pl.BlockSpec((_RB, 128), lambda i, c: (i, 0)),
            pl.BlockSpec((128, 128), lambda i, c: (0, c)),
            pl.BlockSpec((128, 128), lambda i, c: (0, c)),
        ],
        out_specs=[
            pl.BlockSpec((_RB, 128), lambda i, c: (c * (N // _RB) + i, 0)),
            pl.BlockSpec((_RB, 128), lambda i, c: (c * (N // _RB) + i, 0)),
        ],
        out_shape=[
            jax.ShapeDtypeStruct((2 * N, 128), jnp.float32),
            jax.ShapeDtypeStruct((2 * N, 128), jnp.float32),
        ],
    )(x, W1s, W1d)


def _proj2(msg, den, b1, W2s, W2d):
    nb = N // _RB

    def body(msg_ref, den_ref, b1_ref, ws_ref, wd_ref, fs_ref, fd_ref):
        m = jnp.concatenate([msg_ref[0], msg_ref[1]], axis=1)
        dh = [jnp.sum(den_ref[:, h * 16:(h + 1) * 16], axis=1, keepdims=True)
              for h in range(4)]
        col = lax.broadcasted_iota(jnp.int32, (_RB, 256), 1)
        dfull = jnp.where(col < 64, dh[0],
                          jnp.where(col < 128, dh[1],
                                    jnp.where(col < 192, dh[2], dh[3])))
        dfull = jnp.where(dfull > 0, dfull, 1.0)
        h1 = m / dfull + b1_ref[...]
        fs_ref[...] = jnp.dot(h1, ws_ref[...], preferred_element_type=jnp.float32)
        fd_ref[...] = jnp.dot(h1, wd_ref[...], preferred_element_type=jnp.float32)

    return pl.pallas_call(
        body,
        grid=(nb,),
        in_specs=[
            pl.BlockSpec((2, _RB, 128), lambda i: (0, i, 0)),
            pl.BlockSpec((_RB, 64), lambda i: (i, 0)),
            pl.BlockSpec((1, 256), lambda i: (0, 0)),
            pl.BlockSpec((256, 128), lambda i: (0, 0)),
            pl.BlockSpec((256, 128), lambda i: (0, 0)),
        ],
        out_specs=[
            pl.BlockSpec((_RB, 128), lambda i: (i, 0)),
            pl.BlockSpec((_RB, 128), lambda i: (i, 0)),
        ],
        out_shape=[
            jax.ShapeDtypeStruct((N, 128), jnp.float32),
            jax.ShapeDtypeStruct((N, 128), jnp.float32),
        ],
    )(msg, den, b1, W2s, W2d)


def _readout(msg2, den2, b2, Wr1, br1, Wr2, br2):
    nb = N // _RB

    def body(msg_ref, den_ref, b2_ref, wr1_ref, br1_ref, wr2_ref, br2_ref,
             out_ref, acc_ref):
        i = pl.program_id(0)

        @pl.when(i == 0)
        def _():
            acc_ref[...] = jnp.zeros_like(acc_ref)

        m = msg_ref[0, :, 0:64] + msg_ref[1, :, 0:64]
        d = jnp.sum(den_ref[...], axis=1, keepdims=True)
        d = jnp.where(d > 0, d, 1.0)
        h2 = m / d + b2_ref[...]
        acc_ref[...] += jnp.sum(h2, axis=0, keepdims=True)

        @pl.when(i == nb - 1)
        def _():
            hg = acc_ref[...] / jnp.float32(N)
            t = jnp.maximum(
                jnp.dot(hg, wr1_ref[...], preferred_element_type=jnp.float32)
                + br1_ref[...], 0.0)
            out_ref[...] = (jnp.dot(t, wr2_ref[...],
                                    preferred_element_type=jnp.float32)
                            + br2_ref[...])

    return pl.pallas_call(
        body,
        grid=(nb,),
        in_specs=[
            pl.BlockSpec((2, _RB, 128), lambda i: (0, i, 0)),
            pl.BlockSpec((_RB, 32), lambda i: (i, 0)),
            pl.BlockSpec((1, 64), lambda i: (0, 0)),
            pl.BlockSpec((64, 64), lambda i: (0, 0)),
            pl.BlockSpec((1, 64), lambda i: (0, 0)),
            pl.BlockSpec((64, 1), lambda i: (0, 0)),
            pl.BlockSpec((1, 1), lambda i: (0, 0)),
        ],
        out_specs=pl.BlockSpec((1, 1), lambda i: (0, 0)),
        out_shape=jax.ShapeDtypeStruct((1, 1), jnp.float32),
        scratch_shapes=[pltpu.VMEM((1, 64), jnp.float32)],
    )(msg2, den2, b2, Wr1, br1, Wr2, br2)


def kernel(x, W1s, W1d, a1, b1, W2s, W2d, a2, b2, Wr1, br1, Wr2, br2,
           edge_index):
    src = edge_index[0]
    dst = edge_index[1]
    fs1, fd1 = _proj1(x, W1s, W1d)
    msg1, den1 = _make_edge_kernel(2, True)(fs1, fd1, src, dst, a1.reshape(-1))
    den1_t = den1.reshape(NCORE, NSUB, 2, N).transpose(3, 0, 2, 1)
    den1_t = den1_t.reshape(N, NCORE * 2 * NSUB)
    w2s_pad = jnp.pad(W2s, ((0, 0), (0, 64)))
    w2d_pad = jnp.pad(W2d, ((0, 0), (0, 64)))
    fs2, fd2 = _proj2(msg1, den1_t, b1.reshape(1, -1), w2s_pad, w2d_pad)
    msg2, den2 = _make_edge_kernel(1, False)(fs2, fd2, src, dst, a2.reshape(-1))
    den2_t = den2.reshape(NCORE, NSUB, 1, N).transpose(3, 0, 2, 1)
    den2_t = den2_t.reshape(N, NCORE * NSUB)
    out = _readout(msg2, den2_t, b2.reshape(1, -1), Wr1, br1.reshape(1, -1),
                   Wr2, br2.reshape(1, 1))
    return out.reshape(())


# speedup vs baseline: 16.4116x; 1.0061x over previous
import functools

import jax
import jax.numpy as jnp
from jax import lax
from jax.experimental import pallas as pl
from jax.experimental.pallas import tpu as pltpu
from jax.experimental.pallas import tpu_sc as plsc

N = 10000
E = 160000
B = 64
NSUB = 16
NCORE = 2
NBLK = E // B
RCH = 40
NRCH = N // RCH

@functools.lru_cache(maxsize=None)
def _mesh():
    return plsc.VectorSubcoreMesh(
        core_axis_name="c", subcore_axis_name="s",
        num_cores=NCORE, num_subcores=NSUB)


@functools.lru_cache(maxsize=None)
def _make_edge_kernel(nheads, stacked):
    nch = 8
    alen = 256 if stacked else 64

    @functools.partial(
        pl.kernel,
        out_type=(
            jax.ShapeDtypeStruct((NCORE, N, 128), jnp.float32),
            jax.ShapeDtypeStruct((NCORE * NSUB * nheads * N,), jnp.float32),
        ),
        mesh=_mesh(),
        compiler_params=pltpu.CompilerParams(needs_layout_passes=False),
        scratch_types=(
            pltpu.VMEM((B,), jnp.int32),
            pltpu.VMEM((B,), jnp.int32),
            pltpu.VMEM((B,), jnp.int32),
            pltpu.VMEM((B + 16,), jnp.int32),
            pltpu.VMEM((B, 128), jnp.float32),
            pltpu.VMEM((B, 128), jnp.float32),
            pltpu.VMEM((B, 16), jnp.float32),
            pltpu.VMEM((nheads * N + 16,), jnp.float32),
            pltpu.VMEM((alen,), jnp.float32),
            pltpu.SemaphoreType.DMA,
            pltpu.SemaphoreType.DMA,
            pltpu.VMEM_SHARED((N, 128), jnp.float32),
        ),
    )
    def k(fs_hbm, fd_hbm, src_hbm, dst_hbm, a_hbm, msg_out, den_out,
          sidx, didx, didx2, didxp, fsrows, fdrows, exbuf, denloc,
          avm, sem1, sem2, msg_sh):
        c = lax.axis_index("c")
        s = lax.axis_index("s")
        zero = jnp.zeros((16,), jnp.float32)
        lane = lax.iota(jnp.int32, 16)

        def zb(r, carry):
            for kk in range(nch):
                fdrows[r, pl.ds(kk * 16, 16)] = zero
            return carry
        lax.fori_loop(0, B, zb, 0)

        def zd(r, carry):
            denloc[pl.ds(r * 16, 16)] = zero
            return carry
        lax.fori_loop(0, nheads * N // 16 + 1, zd, 0)
        cntz = (NRCH - s + NSUB - 1) // NSUB

        def zcopy(t, carry):
            o = pl.multiple_of((s + t * NSUB) * RCH, 8)
            pltpu.sync_copy(fdrows.at[pl.ds(0, RCH)],
                            msg_sh.at[pl.ds(o, RCH)])
            return carry
        lax.fori_loop(0, cntz, zcopy, 0)

        pltpu.sync_copy(a_hbm, avm)
        avs = []
        for kk in range(4 * nheads):
            if stacked:
                lo = avm[pl.ds(kk * 16, 16)]
                hi = avm[pl.ds(128 + kk * 16, 16)]
                avs.append(jnp.where(c == 0, lo, hi))
            else:
                avs.append(avm[pl.ds(kk * 16, 16)])

        plsc.subcore_barrier()

        if stacked:
            cnt = (NBLK - s + NSUB - 1) // NSUB
        else:
            half = NBLK // NCORE
            cnt = (half - s + NSUB - 1) // NSUB

        def blk_body(i, carry):
            if stacked:
                blk = s + i * NSUB
            else:
                blk = c * (NBLK // NCORE) + s + i * NSUB
            base = pl.multiple_of(blk * B, B)
            pltpu.sync_copy(src_hbm.at[pl.ds(base, B)], sidx)
            pltpu.sync_copy(dst_hbm.at[pl.ds(base, B)], didx)
            if stacked:
                off = c * N
                for kk in range(B // 16):
                    sidx[pl.ds(kk * 16, 16)] = sidx[pl.ds(kk * 16, 16)] + off
                    didx2[pl.ds(kk * 16, 16)] = didx[pl.ds(kk * 16, 16)] + off
                pltpu.async_copy(fs_hbm.at[sidx], fsrows, sem1).wait()
                pltpu.async_copy(fd_hbm.at[didx2], fdrows, sem2).wait()
            else:
                pltpu.async_copy(fs_hbm.at[sidx], fsrows, sem1).wait()
                pltpu.async_copy(fd_hbm.at[didx], fdrows, sem2).wait()
            for kk in range(B // 16):
                didxp[pl.ds(kk * 16, 16)] = didx[pl.ds(kk * 16, 16)]

            def ebody(r):
                fsl = [fsrows[r, pl.ds(kk * 16, 16)] for kk in range(nch)]
                fdl = [fdrows[r, pl.ds(kk * 16, 16)]
                       for kk in range(4 * nheads)]
                exvs = []
                for j in range(nheads):
                    acc = zero
                    for kk in range(4):
                        q = j * 4 + kk
                        ev = fsl[q] + fdl[q]
                        ev = jnp.maximum(ev, 0.2 * ev)
                        acc = acc + ev * avs[q]
                    sco = jnp.sum(acc)
                    exv = jnp.exp(jnp.full((16,), sco, jnp.float32))
                    exvs.append(exv)
                for q in range(nch):
                    fdrows[r, pl.ds(q * 16, 16)] = (
                        fsl[q] * exvs[min(q // 4, nheads - 1)])
                if nheads == 2:
                    exbuf[r, :] = jnp.where(lane == 0, exvs[0],
                                            jnp.where(lane == 1, exvs[1],
                                                      zero))
                else:
                    exbuf[r, :] = jnp.where(lane == 0, exvs[0], zero)
            plsc.parallel_loop(0, B, 1, unroll=4)(ebody)

            def dbody(r, ecarry):
                di = didxp[pl.ds(r, 16)][0]
                exrow = exbuf[r, :]
                for j in range(nheads):
                    p = j * N + di - j
                    dv = denloc[pl.ds(p, 16)]
                    denloc[pl.ds(p, 16)] = dv + jnp.where(lane == j, exrow,
                                                          zero)
                return ecarry
            lax.fori_loop(0, B, dbody, 0)

            pltpu.sync_copy(fdrows, msg_sh.at[didx], add=True)
            return carry
        lax.fori_loop(0, cnt, blk_body, 0)

        doff = pl.multiple_of((c * NSUB + s) * nheads * N, 8)
        pltpu.sync_copy(denloc.at[pl.ds(0, nheads * N)],
                        den_out.at[pl.ds(doff, nheads * N)])

        plsc.subcore_barrier()

        def dcopy(t, carry):
            o = pl.multiple_of((s + t * NSUB) * RCH, 8)
            pltpu.sync_copy(msg_sh.at[pl.ds(o, RCH)], fdrows.at[pl.ds(0, RCH)])
            pltpu.sync_copy(fdrows.at[pl.ds(0, RCH)],
                            msg_out.at[c, pl.ds(o, RCH)])
            return carry
        lax.fori_loop(0, cntz, dcopy, 0)

    return k


_RB = 400


def _proj1(x, W1s, W1d):
    nb = N // _RB

    def body(x_ref, ws_ref, wd_ref, fs_ref, fd_ref):
        xb = x_ref[...]
        fs_ref[...] = jnp.dot(xb, ws_ref[...], preferred_element_type=jnp.float32)
        fd_ref[...] = jnp.dot(xb, wd_ref[...], preferred_element_type=jnp.float32)

    return pl.pallas_call(
        body,
        grid=(nb, 2),
        in_specs=[
            pl.BlockSpec((_RB, 128), lambda i, c: (i, 0)),
            pl.BlockSpec((128, 128), lambda i, c: (0, c)),
            pl.BlockSpec((128, 128), lambda i, c: (0, c)),
        ],
        out_specs=[
            pl.BlockSpec((_RB, 128), lambda i, c: (c * (N // _RB) + i, 0)),
            pl.BlockSpec((_RB, 128), lambda i, c: (c * (N // _RB) + i, 0)),
        ],
        out_shape=[
            jax.ShapeDtypeStruct((2 * N, 128), jnp.float32),
            jax.ShapeDtypeStruct((2 * N, 128), jnp.float32),
        ],
    )(x, W1s, W1d)


def _proj2(msg, den, b1, W2s, W2d):
    nb = N // _RB

    def body(msg_ref, den_ref, b1_ref, ws_ref, wd_ref, fs_ref, fd_ref):
        m = jnp.concatenate([msg_ref[0], msg_ref[1]], axis=1)
        dh = [jnp.sum(den_ref[:, h * 16:(h + 1) * 16], axis=1, keepdims=True)
              for h in range(4)]
        col = lax.broadcasted_iota(jnp.int32, (_RB, 256), 1)
        dfull = jnp.where(col < 64, dh[0],
                          jnp.where(col < 128, dh[1],
                                    jnp.where(col < 192, dh[2], dh[3])))
        dfull = jnp.where(dfull > 0, dfull, 1.0)
        h1 = m / dfull + b1_ref[...]
        fs_ref[...] = jnp.dot(h1, ws_ref[...], preferred_element_type=jnp.float32)
        fd_ref[...] = jnp.dot(h1, wd_ref[...], preferred_element_type=jnp.float32)

    return pl.pallas_call(
        body,
        grid=(nb,),
        in_specs=[
            pl.BlockSpec((2, _RB, 128), lambda i: (0, i, 0)),
            pl.BlockSpec((_RB, 64), lambda i: (i, 0)),
            pl.BlockSpec((1, 256), lambda i: (0, 0)),
            pl.BlockSpec((256, 128), lambda i: (0, 0)),
            pl.BlockSpec((256, 128), lambda i: (0, 0)),
        ],
        out_specs=[
            pl.BlockSpec((_RB, 128), lambda i: (i, 0)),
            pl.BlockSpec((_RB, 128), lambda i: (i, 0)),
        ],
        out_shape=[
            jax.ShapeDtypeStruct((N, 128), jnp.float32),
            jax.ShapeDtypeStruct((N, 128), jnp.float32),
        ],
    )(msg, den, b1, W2s, W2d)


def _readout(msg2, den2, b2, Wr1, br1, Wr2, br2):
    nb = N // _RB

    def body(msg_ref, den_ref, b2_ref, wr1_ref, br1_ref, wr2_ref, br2_ref,
             out_ref, acc_ref):
        i = pl.program_id(0)

        @pl.when(i == 0)
        def _():
            acc_ref[...] = jnp.zeros_like(acc_ref)

        m = msg_ref[0, :, 0:64] + msg_ref[1, :, 0:64]
        d = jnp.sum(den_ref[...], axis=1, keepdims=True)
        d = jnp.where(d > 0, d, 1.0)
        h2 = m / d + b2_ref[...]
        acc_ref[...] += jnp.sum(h2, axis=0, keepdims=True)

        @pl.when(i == nb - 1)
        def _():
            hg = acc_ref[...] / jnp.float32(N)
            t = jnp.maximum(
                jnp.dot(hg, wr1_ref[...], preferred_element_type=jnp.float32)
                + br1_ref[...], 0.0)
            out_ref[...] = (jnp.dot(t, wr2_ref[...],
                                    preferred_element_type=jnp.float32)
                            + br2_ref[...])

    return pl.pallas_call(
        body,
        grid=(nb,),
        in_specs=[
            pl.BlockSpec((2, _RB, 128), lambda i: (0, i, 0)),
            pl.BlockSpec((_RB, 32), lambda i: (i, 0)),
            pl.BlockSpec((1, 64), lambda i: (0, 0)),
            pl.BlockSpec((64, 64), lambda i: (0, 0)),
            pl.BlockSpec((1, 64), lambda i: (0, 0)),
            pl.BlockSpec((64, 1), lambda i: (0, 0)),
            pl.BlockSpec((1, 1), lambda i: (0, 0)),
        ],
        out_specs=pl.BlockSpec((1, 1), lambda i: (0, 0)),
        out_shape=jax.ShapeDtypeStruct((1, 1), jnp.float32),
        scratch_shapes=[pltpu.VMEM((1, 64), jnp.float32)],
    )(msg2, den2, b2, Wr1, br1, Wr2, br2)


def kernel(x, W1s, W1d, a1, b1, W2s, W2d, a2, b2, Wr1, br1, Wr2, br2,
           edge_index):
    src = edge_index[0]
    dst = edge_index[1]
    fs1, fd1 = _proj1(x, W1s, W1d)
    msg1, den1 = _make_edge_kernel(2, True)(fs1, fd1, src, dst, a1.reshape(-1))
    den1_t = den1.reshape(NCORE, NSUB, 2, N).transpose(3, 0, 2, 1)
    den1_t = den1_t.reshape(N, NCORE * 2 * NSUB)
    w2s_pad = jnp.pad(W2s, ((0, 0), (0, 64)))
    w2d_pad = jnp.pad(W2d, ((0, 0), (0, 64)))
    fs2, fd2 = _proj2(msg1, den1_t, b1.reshape(1, -1), w2s_pad, w2d_pad)
    msg2, den2 = _make_edge_kernel(1, False)(fs2, fd2, src, dst, a2.reshape(-1))
    den2_t = den2.reshape(NCORE, NSUB, 1, N).transpose(3, 0, 2, 1)
    den2_t = den2_t.reshape(N, NCORE * NSUB)
    out = _readout(msg2, den2_t, b2.reshape(1, -1), Wr1, br1.reshape(1, -1),
                   Wr2, br2.reshape(1, 1))
    return out.reshape(())


# speedup vs baseline: 20.3325x; 1.2389x over previous
import functools

import jax
import jax.numpy as jnp
from jax import lax
from jax.experimental import pallas as pl
from jax.experimental.pallas import tpu as pltpu
from jax.experimental.pallas import tpu_sc as plsc

N = 10000
E = 160000
B = 64
NSUB = 16
NCORE = 2
NBLK = E // B
RCH = 40
NRCH = N // RCH

@functools.lru_cache(maxsize=None)
def _mesh():
    return plsc.VectorSubcoreMesh(
        core_axis_name="c", subcore_axis_name="s",
        num_cores=NCORE, num_subcores=NSUB)


@functools.lru_cache(maxsize=None)
def _make_edge_kernel(nheads, stacked):
    nch = 8
    alen = 256 if stacked else 64

    @functools.partial(
        pl.kernel,
        out_type=(
            jax.ShapeDtypeStruct((NCORE, N, 128), jnp.float32),
            jax.ShapeDtypeStruct((NCORE * NSUB * nheads * N,), jnp.float32),
        ),
        mesh=_mesh(),
        compiler_params=pltpu.CompilerParams(needs_layout_passes=False),
        scratch_types=(
            pltpu.VMEM((B,), jnp.int32),
            pltpu.VMEM((B,), jnp.int32),
            pltpu.VMEM((B,), jnp.int32),
            pltpu.VMEM((B + 16,), jnp.int32),
            pltpu.VMEM((B, 128), jnp.float32),
            pltpu.VMEM((B, 128), jnp.float32),
            pltpu.VMEM((B, 16), jnp.float32),
            pltpu.VMEM((nheads * N + 16,), jnp.float32),
            pltpu.VMEM((alen,), jnp.float32),
            pltpu.SemaphoreType.DMA,
            pltpu.SemaphoreType.DMA,
            pltpu.VMEM_SHARED((N, 128), jnp.float32),
        ),
    )
    def k(fs_hbm, fd_hbm, src_hbm, dst_hbm, a_hbm, msg_out, den_out,
          sidx, didx, didx2, didxp, fsrows, fdrows, exbuf, denloc,
          avm, sem1, sem2, msg_sh):
        c = lax.axis_index("c")
        s = lax.axis_index("s")
        zero = jnp.zeros((16,), jnp.float32)
        lane = lax.iota(jnp.int32, 16)

        def zb(r, carry):
            for kk in range(nch):
                fdrows[r, pl.ds(kk * 16, 16)] = zero
            return carry
        lax.fori_loop(0, B, zb, 0)

        def zd(r, carry):
            denloc[pl.ds(r * 16, 16)] = zero
            return carry
        lax.fori_loop(0, nheads * N // 16 + 1, zd, 0)
        cntz = (NRCH - s + NSUB - 1) // NSUB

        def zcopy(t, carry):
            o = pl.multiple_of((s + t * NSUB) * RCH, 8)
            pltpu.sync_copy(fdrows.at[pl.ds(0, RCH)],
                            msg_sh.at[pl.ds(o, RCH)])
            return carry
        lax.fori_loop(0, cntz, zcopy, 0)

        pltpu.sync_copy(a_hbm, avm)
        avs = []
        for kk in range(4 * nheads):
            if stacked:
                lo = avm[pl.ds(kk * 16, 16)]
                hi = avm[pl.ds(128 + kk * 16, 16)]
                avs.append(jnp.where(c == 0, lo, hi))
            else:
                avs.append(avm[pl.ds(kk * 16, 16)])

        plsc.subcore_barrier()

        if stacked:
            cnt = (NBLK - s + NSUB - 1) // NSUB
        else:
            half = NBLK // NCORE
            cnt = (half - s + NSUB - 1) // NSUB

        def blk_body(i, carry):
            if stacked:
                blk = s + i * NSUB
            else:
                blk = c * (NBLK // NCORE) + s + i * NSUB
            base = pl.multiple_of(blk * B, B)
            i1 = pltpu.async_copy(src_hbm.at[pl.ds(base, B)], sidx, sem1)
            i2 = pltpu.async_copy(dst_hbm.at[pl.ds(base, B)], didx, sem2)
            i1.wait()
            i2.wait()
            if stacked:
                off = c * N
                for kk in range(B // 16):
                    sidx[pl.ds(kk * 16, 16)] = sidx[pl.ds(kk * 16, 16)] + off
                    didx2[pl.ds(kk * 16, 16)] = didx[pl.ds(kk * 16, 16)] + off
                g1 = pltpu.async_copy(fs_hbm.at[sidx], fsrows, sem1)
                g2 = pltpu.async_copy(fd_hbm.at[didx2], fdrows, sem2)
            else:
                g1 = pltpu.async_copy(fs_hbm.at[sidx], fsrows, sem1)
                g2 = pltpu.async_copy(fd_hbm.at[didx], fdrows, sem2)
            for kk in range(B // 16):
                didxp[pl.ds(kk * 16, 16)] = didx[pl.ds(kk * 16, 16)]
            g1.wait()
            g2.wait()

            def ebody(r):
                fsl = [fsrows[r, pl.ds(kk * 16, 16)] for kk in range(nch)]
                fdl = [fdrows[r, pl.ds(kk * 16, 16)]
                       for kk in range(4 * nheads)]
                exvs = []
                for j in range(nheads):
                    acc = zero
                    for kk in range(4):
                        q = j * 4 + kk
                        ev = fsl[q] + fdl[q]
                        ev = jnp.maximum(ev, 0.2 * ev)
                        acc = acc + ev * avs[q]
                    sco = jnp.sum(acc)
                    exv = jnp.exp(jnp.full((16,), sco, jnp.float32))
                    exvs.append(exv)
                for q in range(nch):
                    fdrows[r, pl.ds(q * 16, 16)] = (
                        fsl[q] * exvs[min(q // 4, nheads - 1)])
                if nheads == 2:
                    exbuf[r, :] = jnp.where(lane == 0, exvs[0],
                                            jnp.where(lane == 1, exvs[1],
                                                      zero))
                else:
                    exbuf[r, :] = jnp.where(lane == 0, exvs[0], zero)
            plsc.parallel_loop(0, B, 1, unroll=4)(ebody)

            def dbody(r, ecarry):
                di = didxp[pl.ds(r, 16)][0]
                exrow = exbuf[r, :]
                for j in range(nheads):
                    p = j * N + di - j
                    dv = denloc[pl.ds(p, 16)]
                    denloc[pl.ds(p, 16)] = dv + jnp.where(lane == j, exrow,
                                                          zero)
                return ecarry
            lax.fori_loop(0, B, dbody, 0)

            pltpu.sync_copy(fdrows, msg_sh.at[didx], add=True)
            return carry
        lax.fori_loop(0, cnt, blk_body, 0)

        doff = pl.multiple_of((c * NSUB + s) * nheads * N, 8)
        pltpu.sync_copy(denloc.at[pl.ds(0, nheads * N)],
                        den_out.at[pl.ds(doff, nheads * N)])

        plsc.subcore_barrier()

        def dcopy(t, carry):
            o = pl.multiple_of((s + t * NSUB) * RCH, 8)
            pltpu.sync_copy(msg_sh.at[pl.ds(o, RCH)], fdrows.at[pl.ds(0, RCH)])
            pltpu.sync_copy(fdrows.at[pl.ds(0, RCH)],
                            msg_out.at[c, pl.ds(o, RCH)])
            return carry
        lax.fori_loop(0, cntz, dcopy, 0)

    return k


_RB = 400


def _proj1(x, W1s, W1d):
    nb = N // _RB

    def body(x_ref, ws_ref, wd_ref, fs_ref, fd_ref):
        xb = x_ref[...]
        fs_ref[...] = jnp.dot(xb, ws_ref[...], preferred_element_type=jnp.float32)
        fd_ref[...] = jnp.dot(xb, wd_ref[...], preferred_element_type=jnp.float32)

    return pl.pallas_call(
        body,
        grid=(nb, 2),
        in_specs=[
            pl.BlockSpec((_RB, 128), lambda i, c: (i, 0)),
            pl.BlockSpec((128, 128), lambda i, c: (0, c)),
            pl.BlockSpec((128, 128), lambda i, c: (0, c)),
        ],
        out_specs=[
            pl.BlockSpec((_RB, 128), lambda i, c: (c * (N // _RB) + i, 0)),
            pl.BlockSpec((_RB, 128), lambda i, c: (c * (N // _RB) + i, 0)),
        ],
        out_shape=[
            jax.ShapeDtypeStruct((2 * N, 128), jnp.float32),
            jax.ShapeDtypeStruct((2 * N, 128), jnp.float32),
        ],
    )(x, W1s, W1d)


def _proj2(msg, den, b1, W2s, W2d):
    nb = N // _RB

    def body(msg_ref, den_ref, b1_ref, ws_ref, wd_ref, fs_ref, fd_ref):
        m = jnp.concatenate([msg_ref[0], msg_ref[1]], axis=1)
        dh = [jnp.sum(den_ref[:, h * 16:(h + 1) * 16], axis=1, keepdims=True)
              for h in range(4)]
        col = lax.broadcasted_iota(jnp.int32, (_RB, 256), 1)
        dfull = jnp.where(col < 64, dh[0],
                          jnp.where(col < 128, dh[1],
                                    jnp.where(col < 192, dh[2], dh[3])))
        dfull = jnp.where(dfull > 0, dfull, 1.0)
        h1 = m / dfull + b1_ref[...]
        fs_ref[...] = jnp.dot(h1, ws_ref[...], preferred_element_type=jnp.float32)
        fd_ref[...] = jnp.dot(h1, wd_ref[...], preferred_element_type=jnp.float32)

    return pl.pallas_call(
        body,
        grid=(nb,),
        in_specs=[
            pl.BlockSpec((2, _RB, 128), lambda i: (0, i, 0)),
            pl.BlockSpec((_RB, 64), lambda i: (i, 0)),
            pl.BlockSpec((1, 256), lambda i: (0, 0)),
            pl.BlockSpec((256, 128), lambda i: (0, 0)),
            pl.BlockSpec((256, 128), lambda i: (0, 0)),
        ],
        out_specs=[
            pl.BlockSpec((_RB, 128), lambda i: (i, 0)),
            pl.BlockSpec((_RB, 128), lambda i: (i, 0)),
        ],
        out_shape=[
            jax.ShapeDtypeStruct((N, 128), jnp.float32),
            jax.ShapeDtypeStruct((N, 128), jnp.float32),
        ],
    )(msg, den, b1, W2s, W2d)


def _readout(msg2, den2, b2, Wr1, br1, Wr2, br2):
    nb = N // _RB

    def body(msg_ref, den_ref, b2_ref, wr1_ref, br1_ref, wr2_ref, br2_ref,
             out_ref, acc_ref):
        i = pl.program_id(0)

        @pl.when(i == 0)
        def _():
            acc_ref[...] = jnp.zeros_like(acc_ref)

        m = msg_ref[0, :, 0:64] + msg_ref[1, :, 0:64]
        d = jnp.sum(den_ref[...], axis=1, keepdims=True)
        d = jnp.where(d > 0, d, 1.0)
        h2 = m / d + b2_ref[...]
        acc_ref[...] += jnp.sum(h2, axis=0, keepdims=True)

        @pl.when(i == nb - 1)
        def _():
            hg = acc_ref[...] / jnp.float32(N)
            t = jnp.maximum(
                jnp.dot(hg, wr1_ref[...], preferred_element_type=jnp.float32)
                + br1_ref[...], 0.0)
            out_ref[...] = (jnp.dot(t, wr2_ref[...],
                                    preferred_element_type=jnp.float32)
                            + br2_ref[...])

    return pl.pallas_call(
        body,
        grid=(nb,),
        in_specs=[
            pl.BlockSpec((2, _RB, 128), lambda i: (0, i, 0)),
            pl.BlockSpec((_RB, 32), lambda i: (i, 0)),
            pl.BlockSpec((1, 64), lambda i: (0, 0)),
            pl.BlockSpec((64, 64), lambda i: (0, 0)),
            pl.BlockSpec((1, 64), lambda i: (0, 0)),
            pl.BlockSpec((64, 1), lambda i: (0, 0)),
            pl.BlockSpec((1, 1), lambda i: (0, 0)),
        ],
        out_specs=pl.BlockSpec((1, 1), lambda i: (0, 0)),
        out_shape=jax.ShapeDtypeStruct((1, 1), jnp.float32),
        scratch_shapes=[pltpu.VMEM((1, 64), jnp.float32)],
    )(msg2, den2, b2, Wr1, br1, Wr2, br2)


def kernel(x, W1s, W1d, a1, b1, W2s, W2d, a2, b2, Wr1, br1, Wr2, br2,
           edge_index):
    src = edge_index[0]
    dst = edge_index[1]
    fs1, fd1 = _proj1(x, W1s, W1d)
    msg1, den1 = _make_edge_kernel(2, True)(fs1, fd1, src, dst, a1.reshape(-1))
    den1_t = den1.reshape(NCORE, NSUB, 2, N).transpose(3, 0, 2, 1)
    den1_t = den1_t.reshape(N, NCORE * 2 * NSUB)
    w2s_pad = jnp.pad(W2s, ((0, 0), (0, 64)))
    w2d_pad = jnp.pad(W2d, ((0, 0), (0, 64)))
    fs2, fd2 = _proj2(msg1, den1_t, b1.reshape(1, -1), w2s_pad, w2d_pad)
    msg2, den2 = _make_edge_kernel(1, False)(fs2, fd2, src, dst, a2.reshape(-1))
    den2_t = den2.reshape(NCORE, NSUB, 1, N).transpose(3, 0, 2, 1)
    den2_t = den2_t.reshape(N, NCORE * NSUB)
    out = _readout(msg2, den2_t, b2.reshape(1, -1), Wr1, br1.reshape(1, -1),
                   Wr2, br2.reshape(1, 1))
    return out.reshape(())


# speedup vs baseline: 22.4608x; 1.1047x over previous
import functools

import jax
import jax.numpy as jnp
from jax import lax
from jax.experimental import pallas as pl
from jax.experimental.pallas import tpu as pltpu
from jax.experimental.pallas import tpu_sc as plsc

N = 10000
E = 160000
B = 32
NSUB = 16
NCORE = 2
NBLK = E // B
RCH = 16
NRCH = N // RCH

@functools.lru_cache(maxsize=None)
def _mesh():
    return plsc.VectorSubcoreMesh(
        core_axis_name="c", subcore_axis_name="s",
        num_cores=NCORE, num_subcores=NSUB)


@functools.lru_cache(maxsize=None)
def _make_edge_kernel(nheads, stacked):
    nch = 8
    alen = 256 if stacked else 64

    @functools.partial(
        pl.kernel,
        out_type=(
            jax.ShapeDtypeStruct((NCORE, N, 128), jnp.float32),
            jax.ShapeDtypeStruct((NCORE * NSUB * nheads * N,), jnp.float32),
        ),
        mesh=_mesh(),
        compiler_params=pltpu.CompilerParams(needs_layout_passes=False),
        scratch_types=(
            pltpu.VMEM((B,), jnp.int32),
            pltpu.VMEM((B,), jnp.int32),
            pltpu.VMEM((B,), jnp.int32),
            pltpu.VMEM((B,), jnp.int32),
            pltpu.VMEM((B,), jnp.int32),
            pltpu.VMEM((B,), jnp.int32),
            pltpu.VMEM((B + 16,), jnp.int32),
            pltpu.VMEM((2, B, 128), jnp.float32),
            pltpu.VMEM((2, B, 128), jnp.float32),
            pltpu.VMEM((B, 16), jnp.float32),
            pltpu.VMEM((nheads * N + 16,), jnp.float32),
            pltpu.VMEM((alen,), jnp.float32),
            pltpu.SemaphoreType.DMA,
            pltpu.SemaphoreType.DMA,
            pltpu.SemaphoreType.DMA,
            pltpu.SemaphoreType.DMA,
            pltpu.VMEM_SHARED((N, 128), jnp.float32),
        ),
    )
    def k(fs_hbm, fd_hbm, src_hbm, dst_hbm, a_hbm, msg_out, den_out,
          sidx0, sidx1, didx0, didx1, didx20, didx21, didxp, fsrows, fdrows,
          exbuf, denloc, avm, semA0, semB0, semA1, semB1, msg_sh):
        sidx = (sidx0, sidx1)
        didx = (didx0, didx1)
        didx2 = (didx20, didx21)
        c = lax.axis_index("c")
        s = lax.axis_index("s")
        zero = jnp.zeros((16,), jnp.float32)
        lane = lax.iota(jnp.int32, 16)

        def zb(r, carry):
            for kk in range(nch):
                fdrows[0, r, pl.ds(kk * 16, 16)] = zero
            return carry
        lax.fori_loop(0, B, zb, 0)

        def zd(r, carry):
            denloc[pl.ds(r * 16, 16)] = zero
            return carry
        lax.fori_loop(0, nheads * N // 16 + 1, zd, 0)
        cntz = (NRCH - s + NSUB - 1) // NSUB

        def zcopy(t, carry):
            o = pl.multiple_of((s + t * NSUB) * RCH, 8)
            pltpu.sync_copy(fdrows.at[0, pl.ds(0, RCH)],
                            msg_sh.at[pl.ds(o, RCH)])
            return carry
        lax.fori_loop(0, cntz, zcopy, 0)

        pltpu.sync_copy(a_hbm, avm)
        avs = []
        for kk in range(4 * nheads):
            if stacked:
                lo = avm[pl.ds(kk * 16, 16)]
                hi = avm[pl.ds(128 + kk * 16, 16)]
                avs.append(jnp.where(c == 0, lo, hi))
            else:
                avs.append(avm[pl.ds(kk * 16, 16)])

        plsc.subcore_barrier()

        if stacked:
            cnt = (NBLK - s + NSUB - 1) // NSUB
        else:
            half = NBLK // NCORE
            cnt = (half - s + NSUB - 1) // NSUB

        sems = ((semA0, semB0), (semA1, semB1))

        def gather_refs(bi):
            if stacked:
                return fs_hbm.at[sidx[bi]], fd_hbm.at[didx2[bi]]
            return fs_hbm.at[sidx[bi]], fd_hbm.at[didx[bi]]

        def start(it, bi):
            if stacked:
                blk = s + it * NSUB
            else:
                blk = c * (NBLK // NCORE) + s + it * NSUB
            base = pl.multiple_of(blk * B, B)
            smA, smB = sems[bi]
            i1 = pltpu.async_copy(src_hbm.at[pl.ds(base, B)], sidx[bi], smA)
            i2 = pltpu.async_copy(dst_hbm.at[pl.ds(base, B)], didx[bi], smB)
            i1.wait()
            i2.wait()
            if stacked:
                off = c * N
                for kk in range(B // 16):
                    sidx[bi][pl.ds(kk * 16, 16)] = (
                        sidx[bi][pl.ds(kk * 16, 16)] + off)
                    didx2[bi][pl.ds(kk * 16, 16)] = (
                        didx[bi][pl.ds(kk * 16, 16)] + off)
            gfs, gfd = gather_refs(bi)
            pltpu.async_copy(gfs, fsrows.at[bi], smA)
            pltpu.async_copy(gfd, fdrows.at[bi], smB)

        def work(bi):
            smA, smB = sems[bi]
            gfs, gfd = gather_refs(bi)
            pltpu.make_async_copy(gfs, fsrows.at[bi], smA).wait()
            pltpu.make_async_copy(gfd, fdrows.at[bi], smB).wait()
            for kk in range(B // 16):
                didxp[pl.ds(kk * 16, 16)] = didx[bi][pl.ds(kk * 16, 16)]

            def ebody(r):
                fsl = [fsrows[bi, r, pl.ds(kk * 16, 16)]
                       for kk in range(nch)]
                fdl = [fdrows[bi, r, pl.ds(kk * 16, 16)]
                       for kk in range(4 * nheads)]
                exvs = []
                for j in range(nheads):
                    acc = zero
                    for kk in range(4):
                        q = j * 4 + kk
                        ev = fsl[q] + fdl[q]
                        ev = jnp.maximum(ev, 0.2 * ev)
                        acc = acc + ev * avs[q]
                    sco = jnp.sum(acc)
                    exv = jnp.exp(jnp.full((16,), sco, jnp.float32))
                    exvs.append(exv)
                for q in range(nch):
                    fdrows[bi, r, pl.ds(q * 16, 16)] = (
                        fsl[q] * exvs[min(q // 4, nheads - 1)])
                if nheads == 2:
                    exbuf[r, :] = jnp.where(lane == 0, exvs[0],
                                            jnp.where(lane == 1, exvs[1],
                                                      zero))
                else:
                    exbuf[r, :] = jnp.where(lane == 0, exvs[0], zero)
            plsc.parallel_loop(0, B, 1, unroll=4)(ebody)

            def dbody(r, ecarry):
                di = didxp[pl.ds(r, 16)][0]
                exrow = exbuf[r, :]
                for j in range(nheads):
                    p = j * N + di - j
                    dv = denloc[pl.ds(p, 16)]
                    denloc[pl.ds(p, 16)] = dv + jnp.where(lane == j, exrow,
                                                          zero)
                return ecarry
            lax.fori_loop(0, B, dbody, 0)

            pltpu.sync_copy(fdrows.at[bi], msg_sh.at[didx[bi]], add=True)

        start(0, 0)

        def pair(p, carry):
            i0 = 2 * p
            i1 = i0 + 1

            @pl.when(i1 < cnt)
            def _():
                start(i1, 1)
            work(0)

            @pl.when(i1 < cnt)
            def _():
                @pl.when(i1 + 1 < cnt)
                def _():
                    start(i1 + 1, 0)
                work(1)
            return carry
        lax.fori_loop(0, (cnt + 1) // 2, pair, 0)

        doff = pl.multiple_of((c * NSUB + s) * nheads * N, 8)
        pltpu.sync_copy(denloc.at[pl.ds(0, nheads * N)],
                        den_out.at[pl.ds(doff, nheads * N)])

        plsc.subcore_barrier()

        def dcopy(t, carry):
            o = pl.multiple_of((s + t * NSUB) * RCH, 8)
            pltpu.sync_copy(msg_sh.at[pl.ds(o, RCH)],
                            fdrows.at[0, pl.ds(0, RCH)])
            pltpu.sync_copy(fdrows.at[0, pl.ds(0, RCH)],
                            msg_out.at[c, pl.ds(o, RCH)])
            return carry
        lax.fori_loop(0, cntz, dcopy, 0)

    return k


_RB = 400


def _proj1(x, W1s, W1d):
    nb = N // _RB

    def body(x_ref, ws_ref, wd_ref, fs_ref, fd_ref):
        xb = x_ref[...]
        fs_ref[...] = jnp.dot(xb, ws_ref[...], preferred_element_type=jnp.float32)
        fd_ref[...] = jnp.dot(xb, wd_ref[...], preferred_element_type=jnp.float32)

    return pl.pallas_call(
        body,
        grid=(nb, 2),
        in_specs=[
            pl.BlockSpec((_RB, 128), lambda i, c: (i, 0)),
            pl.BlockSpec((128, 128), lambda i, c: (0, c)),
            pl.BlockSpec((128, 128), lambda i, c: (0, c)),
        ],
        out_specs=[
            pl.BlockSpec((_RB, 128), lambda i, c: (c * (N // _RB) + i, 0)),
            pl.BlockSpec((_RB, 128), lambda i, c: (c * (N // _RB) + i, 0)),
        ],
        out_shape=[
            jax.ShapeDtypeStruct((2 * N, 128), jnp.float32),
            jax.ShapeDtypeStruct((2 * N, 128), jnp.float32),
        ],
    )(x, W1s, W1d)


def _proj2(msg, den, b1, W2s, W2d):
    nb = N // _RB

    def body(msg_ref, den_ref, b1_ref, ws_ref, wd_ref, fs_ref, fd_ref):
        m = jnp.concatenate([msg_ref[0], msg_ref[1]], axis=1)
        dh = [jnp.sum(den_ref[:, h * 16:(h + 1) * 16], axis=1, keepdims=True)
              for h in range(4)]
        col = lax.broadcasted_iota(jnp.int32, (_RB, 256), 1)
        dfull = jnp.where(col < 64, dh[0],
                          jnp.where(col < 128, dh[1],
                                    jnp.where(col < 192, dh[2], dh[3])))
        dfull = jnp.where(dfull > 0, dfull, 1.0)
        h1 = m / dfull + b1_ref[...]
        fs_ref[...] = jnp.dot(h1, ws_ref[...], preferred_element_type=jnp.float32)
        fd_ref[...] = jnp.dot(h1, wd_ref[...], preferred_element_type=jnp.float32)

    return pl.pallas_call(
        body,
        grid=(nb,),
        in_specs=[
            pl.BlockSpec((2, _RB, 128), lambda i: (0, i, 0)),
            pl.BlockSpec((_RB, 64), lambda i: (i, 0)),
            pl.BlockSpec((1, 256), lambda i: (0, 0)),
            pl.BlockSpec((256, 128), lambda i: (0, 0)),
            pl.BlockSpec((256, 128), lambda i: (0, 0)),
        ],
        out_specs=[
            pl.BlockSpec((_RB, 128), lambda i: (i, 0)),
            pl.BlockSpec((_RB, 128), lambda i: (i, 0)),
        ],
        out_shape=[
            jax.ShapeDtypeStruct((N, 128), jnp.float32),
            jax.ShapeDtypeStruct((N, 128), jnp.float32),
        ],
    )(msg, den, b1, W2s, W2d)


def _readout(msg2, den2, b2, Wr1, br1, Wr2, br2):
    nb = N // _RB

    def body(msg_ref, den_ref, b2_ref, wr1_ref, br1_ref, wr2_ref, br2_ref,
             out_ref, acc_ref):
        i = pl.program_id(0)

        @pl.when(i == 0)
        def _():
            acc_ref[...] = jnp.zeros_like(acc_ref)

        m = msg_ref[0, :, 0:64] + msg_ref[1, :, 0:64]
        d = jnp.sum(den_ref[...], axis=1, keepdims=True)
        d = jnp.where(d > 0, d, 1.0)
        h2 = m / d + b2_ref[...]
        acc_ref[...] += jnp.sum(h2, axis=0, keepdims=True)

        @pl.when(i == nb - 1)
        def _():
            hg = acc_ref[...] / jnp.float32(N)
            t = jnp.maximum(
                jnp.dot(hg, wr1_ref[...], preferred_element_type=jnp.float32)
                + br1_ref[...], 0.0)
            out_ref[...] = (jnp.dot(t, wr2_ref[...],
                                    preferred_element_type=jnp.float32)
                            + br2_ref[...])

    return pl.pallas_call(
        body,
        grid=(nb,),
        in_specs=[
            pl.BlockSpec((2, _RB, 128), lambda i: (0, i, 0)),
            pl.BlockSpec((_RB, 32), lambda i: (i, 0)),
            pl.BlockSpec((1, 64), lambda i: (0, 0)),
            pl.BlockSpec((64, 64), lambda i: (0, 0)),
            pl.BlockSpec((1, 64), lambda i: (0, 0)),
            pl.BlockSpec((64, 1), lambda i: (0, 0)),
            pl.BlockSpec((1, 1), lambda i: (0, 0)),
        ],
        out_specs=pl.BlockSpec((1, 1), lambda i: (0, 0)),
        out_shape=jax.ShapeDtypeStruct((1, 1), jnp.float32),
        scratch_shapes=[pltpu.VMEM((1, 64), jnp.float32)],
    )(msg2, den2, b2, Wr1, br1, Wr2, br2)


def kernel(x, W1s, W1d, a1, b1, W2s, W2d, a2, b2, Wr1, br1, Wr2, br2,
           edge_index):
    src = edge_index[0]
    dst = edge_index[1]
    fs1, fd1 = _proj1(x, W1s, W1d)
    msg1, den1 = _make_edge_kernel(2, True)(fs1, fd1, src, dst, a1.reshape(-1))
    den1_t = den1.reshape(NCORE, NSUB, 2, N).transpose(3, 0, 2, 1)
    den1_t = den1_t.reshape(N, NCORE * 2 * NSUB)
    w2s_pad = jnp.pad(W2s, ((0, 0), (0, 64)))
    w2d_pad = jnp.pad(W2d, ((0, 0), (0, 64)))
    fs2, fd2 = _proj2(msg1, den1_t, b1.reshape(1, -1), w2s_pad, w2d_pad)
    msg2, den2 = _make_edge_kernel(1, False)(fs2, fd2, src, dst, a2.reshape(-1))
    den2_t = den2.reshape(NCORE, NSUB, 1, N).transpose(3, 0, 2, 1)
    den2_t = den2_t.reshape(N, NCORE * NSUB)
    out = _readout(msg2, den2_t, b2.reshape(1, -1), Wr1, br1.reshape(1, -1),
                   Wr2, br2.reshape(1, 1))
    return out.reshape(())


# speedup vs baseline: 24.7896x; 1.1037x over previous
import functools

import jax
import jax.numpy as jnp
from jax import lax
from jax.experimental import pallas as pl
from jax.experimental.pallas import tpu as pltpu
from jax.experimental.pallas import tpu_sc as plsc

N = 10000
E = 160000
B = 32
NSUB = 16
NCORE = 2
NBLK = E // B
RCH = 16
NRCH = N // RCH

@functools.lru_cache(maxsize=None)
def _mesh():
    return plsc.VectorSubcoreMesh(
        core_axis_name="c", subcore_axis_name="s",
        num_cores=NCORE, num_subcores=NSUB)


@functools.lru_cache(maxsize=None)
def _make_edge_kernel(nheads, stacked):
    nch = 8
    alen = 256 if stacked else 64

    @functools.partial(
        pl.kernel,
        out_type=(
            jax.ShapeDtypeStruct((NCORE, N, 128), jnp.float32),
            jax.ShapeDtypeStruct((NCORE * NSUB * nheads * N,), jnp.float32),
        ),
        mesh=_mesh(),
        compiler_params=pltpu.CompilerParams(needs_layout_passes=False),
        scratch_types=(
            pltpu.VMEM((B,), jnp.int32),
            pltpu.VMEM((B,), jnp.int32),
            pltpu.VMEM((B,), jnp.int32),
            pltpu.VMEM((B,), jnp.int32),
            pltpu.VMEM((B,), jnp.int32),
            pltpu.VMEM((B,), jnp.int32),
            pltpu.VMEM((B + 16,), jnp.int32),
            pltpu.VMEM((2, B, 128), jnp.float32),
            pltpu.VMEM((2, B, 128), jnp.float32),
            pltpu.VMEM((B, 16), jnp.float32),
            pltpu.VMEM((nheads * N + 16,), jnp.float32),
            pltpu.VMEM((alen,), jnp.float32),
            pltpu.SemaphoreType.DMA,
            pltpu.SemaphoreType.DMA,
            pltpu.SemaphoreType.DMA,
            pltpu.SemaphoreType.DMA,
            pltpu.VMEM_SHARED((N, 128), jnp.float32),
        ),
    )
    def k(fs_hbm, fd_hbm, src_hbm, dst_hbm, a_hbm, msg_out, den_out,
          sidx0, sidx1, didx0, didx1, didx20, didx21, didxp, fsrows, fdrows,
          exbuf, denloc, avm, semA0, semB0, semA1, semB1, msg_sh):
        sidx = (sidx0, sidx1)
        didx = (didx0, didx1)
        didx2 = (didx20, didx21)
        c = lax.axis_index("c")
        s = lax.axis_index("s")
        zero = jnp.zeros((16,), jnp.float32)
        lane = lax.iota(jnp.int32, 16)

        def zb(r, carry):
            for kk in range(nch):
                fdrows[0, r, pl.ds(kk * 16, 16)] = zero
            return carry
        lax.fori_loop(0, B, zb, 0)

        def zd(r, carry):
            denloc[pl.ds(r * 16, 16)] = zero
            return carry
        lax.fori_loop(0, nheads * N // 16 + 1, zd, 0)
        cntz = (NRCH - s + NSUB - 1) // NSUB

        def zcopy(t, carry):
            o = pl.multiple_of((s + t * NSUB) * RCH, 8)
            pltpu.sync_copy(fdrows.at[0, pl.ds(0, RCH)],
                            msg_sh.at[pl.ds(o, RCH)])
            return carry
        lax.fori_loop(0, cntz, zcopy, 0)

        pltpu.sync_copy(a_hbm, avm)
        avs = []
        for kk in range(4 * nheads):
            if stacked:
                lo = avm[pl.ds(kk * 16, 16)]
                hi = avm[pl.ds(128 + kk * 16, 16)]
                avs.append(jnp.where(c == 0, lo, hi))
            else:
                avs.append(avm[pl.ds(kk * 16, 16)])

        plsc.subcore_barrier()

        if stacked:
            cnt = (NBLK - s + NSUB - 1) // NSUB
        else:
            half = NBLK // NCORE
            cnt = (half - s + NSUB - 1) // NSUB

        sems = ((semA0, semB0), (semA1, semB1))

        def gather_refs(bi):
            if stacked:
                return fs_hbm.at[sidx[bi]], fd_hbm.at[didx2[bi]]
            return fs_hbm.at[sidx[bi]], fd_hbm.at[didx[bi]]

        def start(it, bi):
            if stacked:
                blk = s + it * NSUB
            else:
                blk = c * (NBLK // NCORE) + s + it * NSUB
            base = pl.multiple_of(blk * B, B)
            smA, smB = sems[bi]
            i1 = pltpu.async_copy(src_hbm.at[pl.ds(base, B)], sidx[bi], smA)
            i2 = pltpu.async_copy(dst_hbm.at[pl.ds(base, B)], didx[bi], smB)
            i1.wait()
            i2.wait()
            if stacked:
                off = c * N
                for kk in range(B // 16):
                    sidx[bi][pl.ds(kk * 16, 16)] = (
                        sidx[bi][pl.ds(kk * 16, 16)] + off)
                    didx2[bi][pl.ds(kk * 16, 16)] = (
                        didx[bi][pl.ds(kk * 16, 16)] + off)
            gfs, gfd = gather_refs(bi)
            pltpu.async_copy(gfs, fsrows.at[bi], smA)
            pltpu.async_copy(gfd, fdrows.at[bi], smB)

        def work(bi):
            smA, smB = sems[bi]
            gfs, gfd = gather_refs(bi)
            pltpu.make_async_copy(gfs, fsrows.at[bi], smA).wait()
            pltpu.make_async_copy(gfd, fdrows.at[bi], smB).wait()
            for kk in range(B // 16):
                didxp[pl.ds(kk * 16, 16)] = didx[bi][pl.ds(kk * 16, 16)]

            def ebody(r):
                fsl = [fsrows[bi, r, pl.ds(kk * 16, 16)]
                       for kk in range(nch)]
                fdl = [fdrows[bi, r, pl.ds(kk * 16, 16)]
                       for kk in range(4 * nheads)]
                exvs = []
                for j in range(nheads):
                    acc = zero
                    for kk in range(4):
                        q = j * 4 + kk
                        ev = fsl[q] + fdl[q]
                        ev = jnp.maximum(ev, 0.2 * ev)
                        acc = acc + ev * avs[q]
                    sco = jnp.sum(acc)
                    exv = jnp.exp(jnp.full((16,), sco, jnp.float32))
                    exvs.append(exv)
                for q in range(nch):
                    fdrows[bi, r, pl.ds(q * 16, 16)] = (
                        fsl[q] * exvs[min(q // 4, nheads - 1)])
                if nheads == 2:
                    exbuf[r, :] = jnp.where(lane == 0, exvs[0],
                                            jnp.where(lane == 1, exvs[1],
                                                      zero))
                else:
                    exbuf[r, :] = jnp.where(lane == 0, exvs[0], zero)
            plsc.parallel_loop(0, B, 1, unroll=4)(ebody)

            sc = pltpu.async_copy(fdrows.at[bi], msg_sh.at[didx[bi]],
                                  smA, add=True)

            def dbody(r, ecarry):
                di = didxp[pl.ds(r, 16)][0]
                exrow = exbuf[r, :]
                for j in range(nheads):
                    p = j * N + di - j
                    dv = denloc[pl.ds(p, 16)]
                    denloc[pl.ds(p, 16)] = dv + jnp.where(lane == j, exrow,
                                                          zero)
                return ecarry
            lax.fori_loop(0, B, dbody, 0)
            sc.wait()

        start(0, 0)

        def pair(p, carry):
            i0 = 2 * p
            i1 = i0 + 1

            @pl.when(i1 < cnt)
            def _():
                start(i1, 1)
            work(0)

            @pl.when(i1 < cnt)
            def _():
                @pl.when(i1 + 1 < cnt)
                def _():
                    start(i1 + 1, 0)
                work(1)
            return carry
        lax.fori_loop(0, (cnt + 1) // 2, pair, 0)

        doff = pl.multiple_of((c * NSUB + s) * nheads * N, 8)
        pltpu.sync_copy(denloc.at[pl.ds(0, nheads * N)],
                        den_out.at[pl.ds(doff, nheads * N)])

        plsc.subcore_barrier()

        def dcopy(t, carry):
            o = pl.multiple_of((s + t * NSUB) * RCH, 8)
            pltpu.sync_copy(msg_sh.at[pl.ds(o, RCH)],
                            fdrows.at[0, pl.ds(0, RCH)])
            pltpu.sync_copy(fdrows.at[0, pl.ds(0, RCH)],
                            msg_out.at[c, pl.ds(o, RCH)])
            return carry
        lax.fori_loop(0, cntz, dcopy, 0)

    return k


_RB = 400


def _proj1(x, W1s, W1d):
    nb = N // _RB

    def body(x_ref, ws_ref, wd_ref, fs_ref, fd_ref):
        xb = x_ref[...]
        fs_ref[...] = jnp.dot(xb, ws_ref[...], preferred_element_type=jnp.float32)
        fd_ref[...] = jnp.dot(xb, wd_ref[...], preferred_element_type=jnp.float32)

    return pl.pallas_call(
        body,
        grid=(nb, 2),
        in_specs=[
            pl.BlockSpec((_RB, 128), lambda i, c: (i, 0)),
            pl.BlockSpec((128, 128), lambda i, c: (0, c)),
            pl.BlockSpec((128, 128), lambda i, c: (0, c)),
        ],
        out_specs=[
            pl.BlockSpec((_RB, 128), lambda i, c: (c * (N // _RB) + i, 0)),
            pl.BlockSpec((_RB, 128), lambda i, c: (c * (N // _RB) + i, 0)),
        ],
        out_shape=[
            jax.ShapeDtypeStruct((2 * N, 128), jnp.float32),
            jax.ShapeDtypeStruct((2 * N, 128), jnp.float32),
        ],
    )(x, W1s, W1d)


def _proj2(msg, den, b1, W2s, W2d):
    nb = N // _RB

    def body(msg_ref, den_ref, b1_ref, ws_ref, wd_ref, fs_ref, fd_ref):
        m = jnp.concatenate([msg_ref[0], msg_ref[1]], axis=1)
        dh = [jnp.sum(den_ref[:, h * 16:(h + 1) * 16], axis=1, keepdims=True)
              for h in range(4)]
        col = lax.broadcasted_iota(jnp.int32, (_RB, 256), 1)
        dfull = jnp.where(col < 64, dh[0],
                          jnp.where(col < 128, dh[1],
                                    jnp.where(col < 192, dh[2], dh[3])))
        dfull = jnp.where(dfull > 0, dfull, 1.0)
        h1 = m / dfull + b1_ref[...]
        fs_ref[...] = jnp.dot(h1, ws_ref[...], preferred_element_type=jnp.float32)
        fd_ref[...] = jnp.dot(h1, wd_ref[...], preferred_element_type=jnp.float32)

    return pl.pallas_call(
        body,
        grid=(nb,),
        in_specs=[
            pl.BlockSpec((2, _RB, 128), lambda i: (0, i, 0)),
            pl.BlockSpec((_RB, 64), lambda i: (i, 0)),
            pl.BlockSpec((1, 256), lambda i: (0, 0)),
            pl.BlockSpec((256, 128), lambda i: (0, 0)),
            pl.BlockSpec((256, 128), lambda i: (0, 0)),
        ],
        out_specs=[
            pl.BlockSpec((_RB, 128), lambda i: (i, 0)),
            pl.BlockSpec((_RB, 128), lambda i: (i, 0)),
        ],
        out_shape=[
            jax.ShapeDtypeStruct((N, 128), jnp.float32),
            jax.ShapeDtypeStruct((N, 128), jnp.float32),
        ],
    )(msg, den, b1, W2s, W2d)


def _readout(msg2, den2, b2, Wr1, br1, Wr2, br2):
    nb = N // _RB

    def body(msg_ref, den_ref, b2_ref, wr1_ref, br1_ref, wr2_ref, br2_ref,
             out_ref, acc_ref):
        i = pl.program_id(0)

        @pl.when(i == 0)
        def _():
            acc_ref[...] = jnp.zeros_like(acc_ref)

        m = msg_ref[0, :, 0:64] + msg_ref[1, :, 0:64]
        d = jnp.sum(den_ref[...], axis=1, keepdims=True)
        d = jnp.where(d > 0, d, 1.0)
        h2 = m / d + b2_ref[...]
        acc_ref[...] += jnp.sum(h2, axis=0, keepdims=True)

        @pl.when(i == nb - 1)
        def _():
            hg = acc_ref[...] / jnp.float32(N)
            t = jnp.maximum(
                jnp.dot(hg, wr1_ref[...], preferred_element_type=jnp.float32)
                + br1_ref[...], 0.0)
            out_ref[...] = (jnp.dot(t, wr2_ref[...],
                                    preferred_element_type=jnp.float32)
                            + br2_ref[...])

    return pl.pallas_call(
        body,
        grid=(nb,),
        in_specs=[
            pl.BlockSpec((2, _RB, 128), lambda i: (0, i, 0)),
            pl.BlockSpec((_RB, 32), lambda i: (i, 0)),
            pl.BlockSpec((1, 64), lambda i: (0, 0)),
            pl.BlockSpec((64, 64), lambda i: (0, 0)),
            pl.BlockSpec((1, 64), lambda i: (0, 0)),
            pl.BlockSpec((64, 1), lambda i: (0, 0)),
            pl.BlockSpec((1, 1), lambda i: (0, 0)),
        ],
        out_specs=pl.BlockSpec((1, 1), lambda i: (0, 0)),
        out_shape=jax.ShapeDtypeStruct((1, 1), jnp.float32),
        scratch_shapes=[pltpu.VMEM((1, 64), jnp.float32)],
    )(msg2, den2, b2, Wr1, br1, Wr2, br2)


def kernel(x, W1s, W1d, a1, b1, W2s, W2d, a2, b2, Wr1, br1, Wr2, br2,
           edge_index):
    src = edge_index[0]
    dst = edge_index[1]
    fs1, fd1 = _proj1(x, W1s, W1d)
    msg1, den1 = _make_edge_kernel(2, True)(fs1, fd1, src, dst, a1.reshape(-1))
    den1_t = den1.reshape(NCORE, NSUB, 2, N).transpose(3, 0, 2, 1)
    den1_t = den1_t.reshape(N, NCORE * 2 * NSUB)
    w2s_pad = jnp.pad(W2s, ((0, 0), (0, 64)))
    w2d_pad = jnp.pad(W2d, ((0, 0), (0, 64)))
    fs2, fd2 = _proj2(msg1, den1_t, b1.reshape(1, -1), w2s_pad, w2d_pad)
    msg2, den2 = _make_edge_kernel(1, False)(fs2, fd2, src, dst, a2.reshape(-1))
    den2_t = den2.reshape(NCORE, NSUB, 1, N).transpose(3, 0, 2, 1)
    den2_t = den2_t.reshape(N, NCORE * NSUB)
    out = _readout(msg2, den2_t, b2.reshape(1, -1), Wr1, br1.reshape(1, -1),
                   Wr2, br2.reshape(1, 1))
    return out.reshape(())


# speedup vs baseline: 26.9275x; 1.0862x over previous
import functools

import jax
import jax.numpy as jnp
from jax import lax
from jax.experimental import pallas as pl
from jax.experimental.pallas import tpu as pltpu
from jax.experimental.pallas import tpu_sc as plsc

N = 10000
E = 160000
B = 32
NSUB = 16
NCORE = 2
NBLK = E // B
RCH = 16
NRCH = N // RCH

@functools.lru_cache(maxsize=None)
def _mesh():
    return plsc.VectorSubcoreMesh(
        core_axis_name="c", subcore_axis_name="s",
        num_cores=NCORE, num_subcores=NSUB)


@functools.lru_cache(maxsize=None)
def _make_edge_kernel(nheads, stacked):
    nch = 8
    alen = 256 if stacked else 64

    @functools.partial(
        pl.kernel,
        out_type=(
            jax.ShapeDtypeStruct((NCORE, N, 128), jnp.float32),
            jax.ShapeDtypeStruct((NCORE * NSUB * nheads * N,), jnp.float32),
        ),
        mesh=_mesh(),
        compiler_params=pltpu.CompilerParams(needs_layout_passes=False),
        scratch_types=(
            pltpu.VMEM((B,), jnp.int32),
            pltpu.VMEM((B,), jnp.int32),
            pltpu.VMEM((B,), jnp.int32),
            pltpu.VMEM((B,), jnp.int32),
            pltpu.VMEM((B,), jnp.int32),
            pltpu.VMEM((B,), jnp.int32),
            pltpu.VMEM((B + 16,), jnp.int32),
            pltpu.VMEM((2, B, 128), jnp.float32),
            pltpu.VMEM((2, B, 128), jnp.float32),
            pltpu.VMEM((B, 16), jnp.float32),
            pltpu.VMEM((nheads * N + 16,), jnp.float32),
            pltpu.VMEM((alen,), jnp.float32),
            pltpu.SemaphoreType.DMA,
            pltpu.SemaphoreType.DMA,
            pltpu.SemaphoreType.DMA,
            pltpu.SemaphoreType.DMA,
            pltpu.VMEM_SHARED((N, 128), jnp.float32),
        ),
    )
    def k(fs_hbm, fd_hbm, src_hbm, dst_hbm, a_hbm, msg_out, den_out,
          sidx0, sidx1, didx0, didx1, didx20, didx21, didxp, fsrows, fdrows,
          exbuf, denloc, avm, semA0, semB0, semA1, semB1, msg_sh):
        sidx = (sidx0, sidx1)
        didx = (didx0, didx1)
        didx2 = (didx20, didx21)
        c = lax.axis_index("c")
        s = lax.axis_index("s")
        zero = jnp.zeros((16,), jnp.float32)
        lane = lax.iota(jnp.int32, 16)

        def zb(r, carry):
            for kk in range(nch):
                fdrows[0, r, pl.ds(kk * 16, 16)] = zero
            return carry
        lax.fori_loop(0, B, zb, 0)

        def zd(r, carry):
            denloc[pl.ds(r * 16, 16)] = zero
            return carry
        lax.fori_loop(0, nheads * N // 16 + 1, zd, 0)
        cntz = (NRCH - s + NSUB - 1) // NSUB

        def zcopy(t, carry):
            o = pl.multiple_of((s + t * NSUB) * RCH, 8)
            pltpu.sync_copy(fdrows.at[0, pl.ds(0, RCH)],
                            msg_sh.at[pl.ds(o, RCH)])
            return carry
        lax.fori_loop(0, cntz, zcopy, 0)

        pltpu.sync_copy(a_hbm, avm)
        avs = []
        for kk in range(4 * nheads):
            if stacked:
                lo = avm[pl.ds(kk * 16, 16)]
                hi = avm[pl.ds(128 + kk * 16, 16)]
                avs.append(jnp.where(c == 0, lo, hi))
            else:
                avs.append(avm[pl.ds(kk * 16, 16)])

        plsc.subcore_barrier()

        if stacked:
            cnt = (NBLK - s + NSUB - 1) // NSUB
        else:
            half = NBLK // NCORE
            cnt = (half - s + NSUB - 1) // NSUB

        sems = ((semA0, semB0), (semA1, semB1))

        def gather_refs(bi):
            if stacked:
                return fs_hbm.at[sidx[bi]], fd_hbm.at[didx2[bi]]
            return fs_hbm.at[sidx[bi]], fd_hbm.at[didx[bi]]

        def start(it, bi):
            if stacked:
                blk = s + it * NSUB
            else:
                blk = c * (NBLK // NCORE) + s + it * NSUB
            base = pl.multiple_of(blk * B, B)
            smA, smB = sems[bi]
            i1 = pltpu.async_copy(src_hbm.at[pl.ds(base, B)], sidx[bi], smA)
            i2 = pltpu.async_copy(dst_hbm.at[pl.ds(base, B)], didx[bi], smB)
            i1.wait()
            i2.wait()
            if stacked:
                off = c * N
                for kk in range(B // 16):
                    sidx[bi][pl.ds(kk * 16, 16)] = (
                        sidx[bi][pl.ds(kk * 16, 16)] + off)
                    didx2[bi][pl.ds(kk * 16, 16)] = (
                        didx[bi][pl.ds(kk * 16, 16)] + off)
            gfs, gfd = gather_refs(bi)
            pltpu.async_copy(gfs, fsrows.at[bi], smA)
            pltpu.async_copy(gfd, fdrows.at[bi], smB)

        def work(bi):
            smA, smB = sems[bi]
            gfs, gfd = gather_refs(bi)
            pltpu.make_async_copy(gfs, fsrows.at[bi], smA).wait()
            pltpu.make_async_copy(gfd, fdrows.at[bi], smB).wait()
            for kk in range(B // 16):
                didxp[pl.ds(kk * 16, 16)] = didx[bi][pl.ds(kk * 16, 16)]

            def ebody(r):
                fsl = [fsrows[bi, r, pl.ds(kk * 16, 16)]
                       for kk in range(nch)]
                fdl = [fdrows[bi, r, pl.ds(kk * 16, 16)]
                       for kk in range(4 * nheads)]
                exvs = []
                for j in range(nheads):
                    acc = zero
                    for kk in range(4):
                        q = j * 4 + kk
                        ev = fsl[q] + fdl[q]
                        ev = jnp.maximum(ev, 0.2 * ev)
                        acc = acc + ev * avs[q]
                    sco = jnp.sum(acc)
                    exv = jnp.exp(jnp.full((16,), sco, jnp.float32))
                    exvs.append(exv)
                for q in range(nch):
                    fdrows[bi, r, pl.ds(q * 16, 16)] = (
                        fsl[q] * exvs[min(q // 4, nheads - 1)])
                if nheads == 2:
                    exbuf[r, :] = jnp.where(lane == 0, exvs[0],
                                            jnp.where(lane == 1, exvs[1],
                                                      zero))
                else:
                    exbuf[r, :] = jnp.where(lane == 0, exvs[0], zero)
            plsc.parallel_loop(0, B, 1, unroll=8)(ebody)

            sc = pltpu.async_copy(fdrows.at[bi], msg_sh.at[didx[bi]],
                                  smA, add=True)

            def dbody(r, ecarry):
                di = didxp[pl.ds(r, 16)][0]
                exrow = exbuf[r, :]
                for j in range(nheads):
                    p = j * N + di - j
                    dv = denloc[pl.ds(p, 16)]
                    denloc[pl.ds(p, 16)] = dv + jnp.where(lane == j, exrow,
                                                          zero)
                return ecarry
            lax.fori_loop(0, B, dbody, 0)
            sc.wait()

        start(0, 0)

        def pair(p, carry):
            i0 = 2 * p
            i1 = i0 + 1

            @pl.when(i1 < cnt)
            def _():
                start(i1, 1)
            work(0)

            @pl.when(i1 < cnt)
            def _():
                @pl.when(i1 + 1 < cnt)
                def _():
                    start(i1 + 1, 0)
                work(1)
            return carry
        lax.fori_loop(0, (cnt + 1) // 2, pair, 0)

        doff = pl.multiple_of((c * NSUB + s) * nheads * N, 8)
        pltpu.sync_copy(denloc.at[pl.ds(0, nheads * N)],
                        den_out.at[pl.ds(doff, nheads * N)])

        plsc.subcore_barrier()

        def dcopy(t, carry):
            o = pl.multiple_of((s + t * NSUB) * RCH, 8)
            pltpu.sync_copy(msg_sh.at[pl.ds(o, RCH)],
                            fdrows.at[0, pl.ds(0, RCH)])
            pltpu.sync_copy(fdrows.at[0, pl.ds(0, RCH)],
                            msg_out.at[c, pl.ds(o, RCH)])
            return carry
        lax.fori_loop(0, cntz, dcopy, 0)

    return k


_RB = 400


def _proj1(x, W1s, W1d):
    nb = N // _RB

    def body(x_ref, ws_ref, wd_ref, fs_ref, fd_ref):
        xb = x_ref[...]
        fs_ref[...] = jnp.dot(xb, ws_ref[...], preferred_element_type=jnp.float32)
        fd_ref[...] = jnp.dot(xb, wd_ref[...], preferred_element_type=jnp.float32)

    return pl.pallas_call(
        body,
        grid=(nb, 2),
        in_specs=[
            pl.BlockSpec((_RB, 128), lambda i, c: (i, 0)),
            pl.BlockSpec((128, 128), lambda i, c: (0, c)),
            pl.BlockSpec((128, 128), lambda i, c: (0, c)),
        ],
        out_specs=[
            pl.BlockSpec((_RB, 128), lambda i, c: (c * (N // _RB) + i, 0)),
            pl.BlockSpec((_RB, 128), lambda i, c: (c * (N // _RB) + i, 0)),
        ],
        out_shape=[
            jax.ShapeDtypeStruct((2 * N, 128), jnp.float32),
            jax.ShapeDtypeStruct((2 * N, 128), jnp.float32),
        ],
    )(x, W1s, W1d)


def _proj2(msg, den, b1, W2s, W2d):
    nb = N // _RB

    def body(msg_ref, den_ref, b1_ref, ws_ref, wd_ref, fs_ref, fd_ref):
        m = jnp.concatenate([msg_ref[0], msg_ref[1]], axis=1)
        dh = [jnp.sum(den_ref[:, h * 16:(h + 1) * 16], axis=1, keepdims=True)
              for h in range(4)]
        col = lax.broadcasted_iota(jnp.int32, (_RB, 256), 1)
        dfull = jnp.where(col < 64, dh[0],
                          jnp.where(col < 128, dh[1],
                                    jnp.where(col < 192, dh[2], dh[3])))
        dfull = jnp.where(dfull > 0, dfull, 1.0)
        h1 = m / dfull + b1_ref[...]
        fs_ref[...] = jnp.dot(h1, ws_ref[...], preferred_element_type=jnp.float32)
        fd_ref[...] = jnp.dot(h1, wd_ref[...], preferred_element_type=jnp.float32)

    return pl.pallas_call(
        body,
        grid=(nb,),
        in_specs=[
            pl.BlockSpec((2, _RB, 128), lambda i: (0, i, 0)),
            pl.BlockSpec((_RB, 64), lambda i: (i, 0)),
            pl.BlockSpec((1, 256), lambda i: (0, 0)),
            pl.BlockSpec((256, 128), lambda i: (0, 0)),
            pl.BlockSpec((256, 128), lambda i: (0, 0)),
        ],
        out_specs=[
            pl.BlockSpec((_RB, 128), lambda i: (i, 0)),
            pl.BlockSpec((_RB, 128), lambda i: (i, 0)),
        ],
        out_shape=[
            jax.ShapeDtypeStruct((N, 128), jnp.float32),
            jax.ShapeDtypeStruct((N, 128), jnp.float32),
        ],
    )(msg, den, b1, W2s, W2d)


def _readout(msg2, den2, b2, Wr1, br1, Wr2, br2):
    nb = N // _RB

    def body(msg_ref, den_ref, b2_ref, wr1_ref, br1_ref, wr2_ref, br2_ref,
             out_ref, acc_ref):
        i = pl.program_id(0)

        @pl.when(i == 0)
        def _():
            acc_ref[...] = jnp.zeros_like(acc_ref)

        m = msg_ref[0, :, 0:64] + msg_ref[1, :, 0:64]
        d = jnp.sum(den_ref[...], axis=1, keepdims=True)
        d = jnp.where(d > 0, d, 1.0)
        h2 = m / d + b2_ref[...]
        acc_ref[...] += jnp.sum(h2, axis=0, keepdims=True)

        @pl.when(i == nb - 1)
        def _():
            hg = acc_ref[...] / jnp.float32(N)
            t = jnp.maximum(
                jnp.dot(hg, wr1_ref[...], preferred_element_type=jnp.float32)
                + br1_ref[...], 0.0)
            out_ref[...] = (jnp.dot(t, wr2_ref[...],
                                    preferred_element_type=jnp.float32)
                            + br2_ref[...])

    return pl.pallas_call(
        body,
        grid=(nb,),
        in_specs=[
            pl.BlockSpec((2, _RB, 128), lambda i: (0, i, 0)),
            pl.BlockSpec((_RB, 32), lambda i: (i, 0)),
            pl.BlockSpec((1, 64), lambda i: (0, 0)),
            pl.BlockSpec((64, 64), lambda i: (0, 0)),
            pl.BlockSpec((1, 64), lambda i: (0, 0)),
            pl.BlockSpec((64, 1), lambda i: (0, 0)),
            pl.BlockSpec((1, 1), lambda i: (0, 0)),
        ],
        out_specs=pl.BlockSpec((1, 1), lambda i: (0, 0)),
        out_shape=jax.ShapeDtypeStruct((1, 1), jnp.float32),
        scratch_shapes=[pltpu.VMEM((1, 64), jnp.float32)],
    )(msg2, den2, b2, Wr1, br1, Wr2, br2)


def kernel(x, W1s, W1d, a1, b1, W2s, W2d, a2, b2, Wr1, br1, Wr2, br2,
           edge_index):
    src = edge_index[0]
    dst = edge_index[1]
    fs1, fd1 = _proj1(x, W1s, W1d)
    msg1, den1 = _make_edge_kernel(2, True)(fs1, fd1, src, dst, a1.reshape(-1))
    den1_t = den1.reshape(NCORE, NSUB, 2, N).transpose(3, 0, 2, 1)
    den1_t = den1_t.reshape(N, NCORE * 2 * NSUB)
    w2s_pad = jnp.pad(W2s, ((0, 0), (0, 64)))
    w2d_pad = jnp.pad(W2d, ((0, 0), (0, 64)))
    fs2, fd2 = _proj2(msg1, den1_t, b1.reshape(1, -1), w2s_pad, w2d_pad)
    msg2, den2 = _make_edge_kernel(1, False)(fs2, fd2, src, dst, a2.reshape(-1))
    den2_t = den2.reshape(NCORE, NSUB, 1, N).transpose(3, 0, 2, 1)
    den2_t = den2_t.reshape(N, NCORE * NSUB)
    out = _readout(msg2, den2_t, b2.reshape(1, -1), Wr1, br1.reshape(1, -1),
                   Wr2, br2.reshape(1, 1))
    return out.reshape(())
